# baseline (speedup 1.0000x reference)
import functools

import numpy as np

import jax
import jax.numpy as jnp
from jax import lax
from jax.experimental import pallas as pl
from jax.experimental.pallas import tpu as pltpu
from jax.experimental.pallas import tpu_sc as plsc

N = 10000
E = 320000
D = 128
NC = 2
NS = 16
NW = NC * NS
EPW = E // NW
C = 40
NCHUNK = EPW // C
PAIRS = NCHUNK // 2
WIDTH = D + 16
RPT = N // NS
NG = D // 16
UE = 5


_PB = np.stack([np.arange(16), np.arange(16) + 16], axis=1).reshape(-1)
PERM = np.concatenate([_PB + 32 * blk for blk in range(D // 32)])


def _proj_body(src_ref, tgt_ref, wst_ref, wtt_ref, b_ref, xs_ref, xt_ref):
    xs_ref[...] = jnp.dot(src_ref[...], wst_ref[...],
                          preferred_element_type=jnp.float32
                          ).astype(jnp.bfloat16)
    xt_ref[...] = (jnp.dot(tgt_ref[...], wtt_ref[...],
                           preferred_element_type=jnp.float32)
                   + b_ref[...]).astype(jnp.bfloat16)


def _sc_body(xs_hbm, xt_hbm, is_hbm, it_hbm, dist_hbm, wd_hbm, zeros_hbm,
             out_hbm,
             idx_s0, idx_t0, sidx0, dist0, rows_s0, rows_t0, msgs0,
             idx_s1, idx_t1, sidx1, dist1, rows_s1, rows_t1, msgs1,
             wd_v, acc_sh, sem_gs0, sem_gt0, sem_gs1, sem_gt1,
             sem_i0, sem_i1, sem_it0, sem_it1, sem_sc0, sem_sc1):
    idx_s_v = (idx_s0, idx_s1)
    idx_t_v = (idx_t0, idx_t1)
    sidx_v = (sidx0, sidx1)
    dist_v = (dist0, dist1)
    rows_s_v = (rows_s0, rows_s1)
    rows_t_v = (rows_t0, rows_t1)
    msgs_v = (msgs0, msgs1)
    c = lax.axis_index("c")
    s = lax.axis_index("s")
    wid = c * NS + s

    one = jnp.ones((16,), jnp.float32)

    pltpu.sync_copy(zeros_hbm.at[pl.ds(s * RPT, RPT)],
                    acc_sh.at[pl.ds(s * RPT, RPT)])

    def _onerow(i, carry):
        msgs_v[0][i, pl.ds(D, 16)] = one
        msgs_v[1][i, pl.ds(D, 16)] = one
        return carry
    lax.fori_loop(0, C, _onerow, 0)

    pltpu.sync_copy(wd_hbm, wd_v)
    wds = [wd_v[pl.ds(jj * 32, 32)] for jj in range(D // 32)]

    plsc.subcore_barrier()

    base = wid * EPW
    sem_g = ((sem_gs0, sem_gt0), (sem_gs1, sem_gt1))
    sem_i = (sem_i0, sem_i1)
    sem_it = (sem_it0, sem_it1)
    sem_sc = (sem_sc0, sem_sc1)

    def isd_copy(k, b):
        off = base + k * C
        pltpu.make_async_copy(is_hbm.at[pl.ds(off, C)],
                              idx_s_v[b], sem_i[b]).start()
        pltpu.make_async_copy(it_hbm.at[pl.ds(off, C)],
                              idx_t_v[b], sem_i[b]).start()
        pltpu.make_async_copy(dist_hbm.at[pl.ds(off, C)],
                              dist_v[b], sem_i[b]).start()

    def isd_wait(k, b):
        off = base + k * C
        pltpu.make_async_copy(is_hbm.at[pl.ds(off, C)],
                              idx_s_v[b], sem_i[b]).wait()
        pltpu.make_async_copy(it_hbm.at[pl.ds(off, C)],
                              idx_t_v[b], sem_i[b]).wait()
        pltpu.make_async_copy(dist_hbm.at[pl.ds(off, C)],
                              dist_v[b], sem_i[b]).wait()

    def sidx_copy(k, b):
        off = base + k * C
        pltpu.make_async_copy(it_hbm.at[pl.ds(off, C)],
                              sidx_v[b], sem_it[b]).start()

    def sidx_wait(k, b):
        off = base + k * C
        pltpu.make_async_copy(it_hbm.at[pl.ds(off, C)],
                              sidx_v[b], sem_it[b]).wait()

    def gather_desc(b):
        return (pltpu.make_async_copy(xs_hbm.at[idx_s_v[b]],
                                      rows_s_v[b], sem_g[b][0]),
                pltpu.make_async_copy(xt_hbm.at[idx_t_v[b]],
                                      rows_t_v[b], sem_g[b][1]))

    def gather_start(b):
        for d in gather_desc(b):
            d.start()

    def gather_wait(b):
        for d in gather_desc(b):
            d.wait()

    def scat_desc(b):
        return pltpu.make_async_copy(msgs_v[b], acc_sh.at[sidx_v[b]],
                                     sem_sc[b])

    def compute(b):
        @plsc.parallel_loop(0, C, 1, unroll=UE)
        def _edge(i):
            db = plsc.load_gather(dist_v[b],
                                  [jnp.full((16,), i, jnp.int32)])
            db2 = plsc.pack(db, db, format=plsc.PackFormat.INTERLEAVED)
            for jj in range(D // 32):
                v = (rows_s_v[b][i, pl.ds(jj * 32, 32)]
                     + rows_t_v[b][i, pl.ds(jj * 32, 32)]
                     + db2 * wds[jj])
                lo, hi = plsc.unpack(v, format=plsc.PackFormat.INTERLEAVED)
                msgs_v[b][i, pl.ds(jj * 32, 16)] = jnp.maximum(lo, 0.0)
                msgs_v[b][i, pl.ds(jj * 32 + 16, 16)] = jnp.maximum(hi, 0.0)

    isd_copy(0, 0)
    isd_copy(1, 1)
    isd_wait(0, 0)
    gather_start(0)

    def _pair(g, carry):
        k0 = 2 * g
        isd_wait(k0 + 1, 1)
        gather_start(1)
        gather_wait(0)

        @pl.when(g > 0)
        def _():
            scat_desc(0).wait()
        sidx_copy(k0, 0)
        compute(0)
        sidx_wait(k0, 0)
        scat_desc(0).start(add=True)

        @pl.when(g < PAIRS - 1)
        def _():
            isd_copy(k0 + 2, 0)

        gather_wait(1)

        @pl.when(g < PAIRS - 1)
        def _():
            isd_copy(k0 + 3, 1)

        @pl.when(g > 0)
        def _():
            scat_desc(1).wait()
        sidx_copy(k0 + 1, 1)

        @pl.when(g < PAIRS - 1)
        def _():
            isd_wait(k0 + 2, 0)
            gather_start(0)
        compute(1)
        sidx_wait(k0 + 1, 1)
        scat_desc(1).start(add=True)
        return carry
    lax.fori_loop(0, PAIRS, _pair, 0)

    scat_desc(0).wait()
    scat_desc(1).wait()

    plsc.subcore_barrier()

    r0 = s * RPT
    pltpu.sync_copy(acc_sh.at[pl.ds(r0, RPT)], out_hbm.at[c, pl.ds(r0, RPT)])


def _post_body(tgt_ref, a0_ref, a1_ref, wrt_ref, wc1t_ref, wc2t_ref, b_ref,
               g_ref, beta_ref, out_ref):
    sums = a0_ref[0, :, :D] + a1_ref[0, :, :D]
    cnt = a0_ref[0, :, D:D + 1] + a1_ref[0, :, D:D + 1]
    aggr = sums / jnp.maximum(cnt, 1.0)
    h = (jnp.dot(tgt_ref[...], wrt_ref[...] + wc1t_ref[...],
                 preferred_element_type=jnp.float32)
         + jnp.dot(aggr, wc2t_ref[...], preferred_element_type=jnp.float32)
         + b_ref[...])
    mean = jnp.mean(h, axis=-1, keepdims=True)
    var = jnp.mean(jnp.square(h - mean), axis=-1, keepdims=True)
    out_ref[...] = ((h - mean) * lax.rsqrt(var + 1e-5) * g_ref[...]
                    + beta_ref[...])


def kernel(source_node, target_node, edge_index, edge_attr, distance,
           W_msg, b_msg, W_res, W_comb, b_comb, ln_gamma, ln_beta):
    del edge_attr
    wst = W_msg[:, :D].T[:, PERM]
    wtt = W_msg[:, D:2 * D].T[:, PERM]
    wd = W_msg[:, 2 * D][PERM].astype(jnp.bfloat16)
    i_s = edge_index[0]
    i_t = edge_index[1]
    dist = distance[:, 0]

    BLK = 10000
    grid = N // BLK
    full = pl.BlockSpec((D, D), lambda i: (0, 0))
    row = pl.BlockSpec((1, D), lambda i: (0, 0))
    nblk = pl.BlockSpec((BLK, D), lambda i: (i, 0))

    xs, xt = pl.pallas_call(
        _proj_body,
        grid=(grid,),
        in_specs=[nblk, nblk, full, full, row],
        out_specs=[nblk, nblk],
        out_shape=[jax.ShapeDtypeStruct((N, D), jnp.bfloat16)] * 2,
    )(source_node, target_node, wst, wtt, b_msg[PERM].reshape(1, D))

    mesh = plsc.VectorSubcoreMesh(core_axis_name="c", subcore_axis_name="s")
    acc = pl.kernel(
        _sc_body,
        out_type=jax.ShapeDtypeStruct((NC, N, WIDTH), jnp.float32),
        mesh=mesh,
        compiler_params=pltpu.CompilerParams(use_tc_tiling_on_sc=False,
                                              needs_layout_passes=False),
        scratch_types=[
            pltpu.VMEM((C,), jnp.int32),
            pltpu.VMEM((C,), jnp.int32),
            pltpu.VMEM((C,), jnp.int32),
            pltpu.VMEM((C,), jnp.float32),
            pltpu.VMEM((C, D), jnp.bfloat16),
            pltpu.VMEM((C, D), jnp.bfloat16),
            pltpu.VMEM((C, WIDTH), jnp.float32),
        ] * 2 + [
            pltpu.VMEM((D,), jnp.bfloat16),
            pltpu.VMEM_SHARED((N, WIDTH), jnp.float32),
        ] + [pltpu.SemaphoreType.DMA] * 10,
    )(xs, xt, i_s, i_t, dist, wd, jnp.zeros((N, WIDTH), jnp.float32))

    a0blk = pl.BlockSpec((1, BLK, WIDTH), lambda i: (0, i, 0))
    a1blk = pl.BlockSpec((1, BLK, WIDTH), lambda i: (1, i, 0))
    out = pl.pallas_call(
        _post_body,
        grid=(grid,),
        in_specs=[nblk, a0blk, a1blk, full, full, full, row, row, row],
        out_specs=nblk,
        out_shape=jax.ShapeDtypeStruct((N, D), jnp.float32),
    )(target_node, acc, acc, W_res.T, W_comb[:, :D].T, W_comb[:, D:].T,
      b_comb.reshape(1, D), ln_gamma.reshape(1, D), ln_beta.reshape(1, D))
    return out


# speedup vs baseline: 1.0030x; 1.0030x over previous
import functools

import numpy as np

import jax
import jax.numpy as jnp
from jax import lax
from jax.experimental import pallas as pl
from jax.experimental.pallas import tpu as pltpu
from jax.experimental.pallas import tpu_sc as plsc

N = 10000
E = 320000
D = 128
NC = 2
NS = 16
NW = NC * NS
EPW = E // NW
C = 40
NCHUNK = EPW // C
PAIRS = NCHUNK // 2
WIDTH = D + 16
RPT = N // NS
NG = D // 16
UE = 4


_PB = np.stack([np.arange(16), np.arange(16) + 16], axis=1).reshape(-1)
PERM = np.concatenate([_PB + 32 * blk for blk in range(D // 32)])


def _proj_body(src_ref, tgt_ref, wst_ref, wtt_ref, b_ref, xs_ref, xt_ref):
    xs_ref[...] = jnp.dot(src_ref[...], wst_ref[...],
                          preferred_element_type=jnp.float32
                          ).astype(jnp.bfloat16)
    xt_ref[...] = (jnp.dot(tgt_ref[...], wtt_ref[...],
                           preferred_element_type=jnp.float32)
                   + b_ref[...]).astype(jnp.bfloat16)


def _sc_body(xs_hbm, xt_hbm, is_hbm, it_hbm, dist_hbm, wd_hbm, zeros_hbm,
             out_hbm,
             idx_s0, idx_t0, sidx0, dist0, rows_s0, rows_t0, msgs0,
             idx_s1, idx_t1, sidx1, dist1, rows_s1, rows_t1, msgs1,
             wd_v, acc_sh, sem_gs0, sem_gt0, sem_gs1, sem_gt1,
             sem_i0, sem_i1, sem_it0, sem_it1, sem_sc0, sem_sc1):
    idx_s_v = (idx_s0, idx_s1)
    idx_t_v = (idx_t0, idx_t1)
    sidx_v = (sidx0, sidx1)
    dist_v = (dist0, dist1)
    rows_s_v = (rows_s0, rows_s1)
    rows_t_v = (rows_t0, rows_t1)
    msgs_v = (msgs0, msgs1)
    c = lax.axis_index("c")
    s = lax.axis_index("s")
    wid = c * NS + s

    one = jnp.ones((16,), jnp.float32)

    pltpu.sync_copy(zeros_hbm.at[pl.ds(s * RPT, RPT)],
                    acc_sh.at[pl.ds(s * RPT, RPT)])

    def _onerow(i, carry):
        msgs_v[0][i, pl.ds(D, 16)] = one
        msgs_v[1][i, pl.ds(D, 16)] = one
        return carry
    lax.fori_loop(0, C, _onerow, 0)

    pltpu.sync_copy(wd_hbm, wd_v)
    wds = [wd_v[pl.ds(jj * 32, 32)] for jj in range(D // 32)]

    plsc.subcore_barrier()

    base = wid * EPW
    sem_g = ((sem_gs0, sem_gt0), (sem_gs1, sem_gt1))
    sem_i = (sem_i0, sem_i1)
    sem_it = (sem_it0, sem_it1)
    sem_sc = (sem_sc0, sem_sc1)

    def isd_copy(k, b):
        off = base + k * C
        pltpu.make_async_copy(is_hbm.at[pl.ds(off, C)],
                              idx_s_v[b], sem_i[b]).start()
        pltpu.make_async_copy(it_hbm.at[pl.ds(off, C)],
                              idx_t_v[b], sem_i[b]).start()
        pltpu.make_async_copy(dist_hbm.at[pl.ds(off, C)],
                              dist_v[b], sem_i[b]).start()

    def isd_wait(k, b):
        off = base + k * C
        pltpu.make_async_copy(is_hbm.at[pl.ds(off, C)],
                              idx_s_v[b], sem_i[b]).wait()
        pltpu.make_async_copy(it_hbm.at[pl.ds(off, C)],
                              idx_t_v[b], sem_i[b]).wait()
        pltpu.make_async_copy(dist_hbm.at[pl.ds(off, C)],
                              dist_v[b], sem_i[b]).wait()

    def sidx_copy(k, b):
        off = base + k * C
        pltpu.make_async_copy(it_hbm.at[pl.ds(off, C)],
                              sidx_v[b], sem_it[b]).start()

    def sidx_wait(k, b):
        off = base + k * C
        pltpu.make_async_copy(it_hbm.at[pl.ds(off, C)],
                              sidx_v[b], sem_it[b]).wait()

    def gather_desc(b):
        return (pltpu.make_async_copy(xs_hbm.at[idx_s_v[b]],
                                      rows_s_v[b], sem_g[b][0]),
                pltpu.make_async_copy(xt_hbm.at[idx_t_v[b]],
                                      rows_t_v[b], sem_g[b][1]))

    def gather_start(b):
        for d in gather_desc(b):
            d.start()

    def gather_wait(b):
        for d in gather_desc(b):
            d.wait()

    def scat_desc(b):
        return pltpu.make_async_copy(msgs_v[b], acc_sh.at[sidx_v[b]],
                                     sem_sc[b])

    def compute(b):
        @plsc.parallel_loop(0, C, 1, unroll=UE)
        def _edge(i):
            db = plsc.load_gather(dist_v[b],
                                  [jnp.full((16,), i, jnp.int32)])
            db2 = plsc.pack(db, db, format=plsc.PackFormat.INTERLEAVED)
            for jj in range(D // 32):
                v = (rows_s_v[b][i, pl.ds(jj * 32, 32)]
                     + rows_t_v[b][i, pl.ds(jj * 32, 32)]
                     + db2 * wds[jj])
                lo, hi = plsc.unpack(v, format=plsc.PackFormat.INTERLEAVED)
                msgs_v[b][i, pl.ds(jj * 32, 16)] = jnp.maximum(lo, 0.0)
                msgs_v[b][i, pl.ds(jj * 32 + 16, 16)] = jnp.maximum(hi, 0.0)

    isd_copy(0, 0)
    isd_copy(1, 1)
    isd_wait(0, 0)
    gather_start(0)

    def _pair(g, carry):
        k0 = 2 * g
        isd_wait(k0 + 1, 1)
        gather_start(1)
        gather_wait(0)

        @pl.when(g > 0)
        def _():
            scat_desc(0).wait()
        sidx_copy(k0, 0)
        compute(0)
        sidx_wait(k0, 0)
        scat_desc(0).start(add=True)

        @pl.when(g < PAIRS - 1)
        def _():
            isd_copy(k0 + 2, 0)

        gather_wait(1)

        @pl.when(g < PAIRS - 1)
        def _():
            isd_copy(k0 + 3, 1)

        @pl.when(g > 0)
        def _():
            scat_desc(1).wait()
        sidx_copy(k0 + 1, 1)

        @pl.when(g < PAIRS - 1)
        def _():
            isd_wait(k0 + 2, 0)
            gather_start(0)
        compute(1)
        sidx_wait(k0 + 1, 1)
        scat_desc(1).start(add=True)
        return carry
    lax.fori_loop(0, PAIRS, _pair, 0)

    scat_desc(0).wait()
    scat_desc(1).wait()

    plsc.subcore_barrier()

    r0 = s * RPT
    pltpu.sync_copy(acc_sh.at[pl.ds(r0, RPT)], out_hbm.at[c, pl.ds(r0, RPT)])


def _post_body(tgt_ref, a0_ref, a1_ref, wrt_ref, wc1t_ref, wc2t_ref, b_ref,
               g_ref, beta_ref, out_ref):
    sums = a0_ref[0, :, :D] + a1_ref[0, :, :D]
    cnt = a0_ref[0, :, D:D + 1] + a1_ref[0, :, D:D + 1]
    aggr = sums / jnp.maximum(cnt, 1.0)
    h = (jnp.dot(tgt_ref[...], wrt_ref[...] + wc1t_ref[...],
                 preferred_element_type=jnp.float32)
         + jnp.dot(aggr, wc2t_ref[...], preferred_element_type=jnp.float32)
         + b_ref[...])
    mean = jnp.mean(h, axis=-1, keepdims=True)
    var = jnp.mean(jnp.square(h - mean), axis=-1, keepdims=True)
    out_ref[...] = ((h - mean) * lax.rsqrt(var + 1e-5) * g_ref[...]
                    + beta_ref[...])


def kernel(source_node, target_node, edge_index, edge_attr, distance,
           W_msg, b_msg, W_res, W_comb, b_comb, ln_gamma, ln_beta):
    del edge_attr
    wst = W_msg[:, :D].T[:, PERM]
    wtt = W_msg[:, D:2 * D].T[:, PERM]
    wd = W_msg[:, 2 * D][PERM].astype(jnp.bfloat16)
    i_s = edge_index[0]
    i_t = edge_index[1]
    dist = distance[:, 0]

    BLK = 10000
    grid = N // BLK
    full = pl.BlockSpec((D, D), lambda i: (0, 0))
    row = pl.BlockSpec((1, D), lambda i: (0, 0))
    nblk = pl.BlockSpec((BLK, D), lambda i: (i, 0))

    xs, xt = pl.pallas_call(
        _proj_body,
        grid=(grid,),
        in_specs=[nblk, nblk, full, full, row],
        out_specs=[nblk, nblk],
        out_shape=[jax.ShapeDtypeStruct((N, D), jnp.bfloat16)] * 2,
    )(source_node, target_node, wst, wtt, b_msg[PERM].reshape(1, D))

    mesh = plsc.VectorSubcoreMesh(core_axis_name="c", subcore_axis_name="s")
    acc = pl.kernel(
        _sc_body,
        out_type=jax.ShapeDtypeStruct((NC, N, WIDTH), jnp.float32),
        mesh=mesh,
        compiler_params=pltpu.CompilerParams(use_tc_tiling_on_sc=False,
                                              needs_layout_passes=False),
        scratch_types=[
            pltpu.VMEM((C,), jnp.int32),
            pltpu.VMEM((C,), jnp.int32),
            pltpu.VMEM((C,), jnp.int32),
            pltpu.VMEM((C,), jnp.float32),
            pltpu.VMEM((C, D), jnp.bfloat16),
            pltpu.VMEM((C, D), jnp.bfloat16),
            pltpu.VMEM((C, WIDTH), jnp.float32),
        ] * 2 + [
            pltpu.VMEM((D,), jnp.bfloat16),
            pltpu.VMEM_SHARED((N, WIDTH), jnp.float32),
        ] + [pltpu.SemaphoreType.DMA] * 10,
    )(xs, xt, i_s, i_t, dist, wd, jnp.zeros((N, WIDTH), jnp.float32))

    a0blk = pl.BlockSpec((1, BLK, WIDTH), lambda i: (0, i, 0))
    a1blk = pl.BlockSpec((1, BLK, WIDTH), lambda i: (1, i, 0))
    out = pl.pallas_call(
        _post_body,
        grid=(grid,),
        in_specs=[nblk, a0blk, a1blk, full, full, full, row, row, row],
        out_specs=nblk,
        out_shape=jax.ShapeDtypeStruct((N, D), jnp.float32),
    )(target_node, acc, acc, W_res.T, W_comb[:, :D].T, W_comb[:, D:].T,
      b_comb.reshape(1, D), ln_gamma.reshape(1, D), ln_beta.reshape(1, D))
    return out
